# baseline (speedup 1.0000x reference)
import functools

import jax
import jax.numpy as jnp
from jax import lax
from jax.experimental import pallas as pl
from jax.experimental.pallas import tpu as pltpu
from jax.experimental.pallas import tpu_sc as plsc

ALPHA_C = 5.0
BETA_C = 0.5
S_SAMPLES = 128
S_SC = 40
TILE_B = 128
_SC_WORKERS = 32
_SC_GRP = 16384

_K0 = 0
_K1 = 42
_K2 = 0x1BD11BDA ^ _K0 ^ _K1


def _rotl(v, r):
    return (v << jnp.uint32(r)) | (v >> jnp.uint32(32 - r))


def _rounds(x0, x1, rots):
    for r in rots:
        x0 = x0 + x1
        x1 = _rotl(x1, r) ^ x0
    return x0, x1


def _threefry_core(x1):
    r0 = (13, 15, 26, 6)
    r1 = (17, 29, 16, 24)
    x0 = x1
    x1 = _rotl(x1, 13) ^ x0
    x0, x1 = _rounds(x0, x1, r0[1:])
    x0 = x0 + jnp.uint32(_K1)
    x1 = x1 + jnp.uint32((_K2 + 1) & 0xFFFFFFFF)
    x0, x1 = _rounds(x0, x1, r1)
    x0 = x0 + jnp.uint32(_K2)
    x1 = x1 + jnp.uint32((_K0 + 2) & 0xFFFFFFFF)
    x0, x1 = _rounds(x0, x1, r0)
    x0 = x0 + jnp.uint32(_K0)
    x1 = x1 + jnp.uint32((_K1 + 3) & 0xFFFFFFFF)
    x0, x1 = _rounds(x0, x1, r1)
    x0 = x0 + jnp.uint32(_K1)
    x1 = x1 + jnp.uint32((_K2 + 4) & 0xFFFFFFFF)
    x0, x1 = _rounds(x0, x1, r0)
    x0 = x0 + jnp.uint32(_K2)
    x1 = x1 + jnp.uint32((_K0 + 5) & 0xFFFFFFFF)
    return x0 ^ x1


def _rank_from_bits(bits, invp):
    fb = (bits >> jnp.uint32(9)) | jnp.uint32(0x3F800000)
    u = jax.lax.bitcast_convert_type(fb, jnp.float32) - 1.0
    return jnp.log(u) * invp


def _sc_bits_kernel(bits_hbm, buf0, buf1, sem0, sem1, *, n_tok, hidden):
    wid = lax.axis_index("s") * 2 + lax.axis_index("c")
    rows_per_w = (S_SC * n_tok) // _SC_WORKERS
    rows_per_grp = _SC_GRP // hidden
    groups = rows_per_w // rows_per_grp
    chunks_per_row = hidden // 16
    w_row = wid * rows_per_w
    lane = lax.iota(jnp.int32, 16).astype(jnp.uint32)
    iv0 = (w_row * hidden).astype(jnp.uint32) + lane + jnp.uint32(_K1)
    bufs = (buf0, buf1)
    sems = (sem0, sem1)

    def fill(buf, iv):
        def chunk_body(j, iv):
            buf[j // chunks_per_row,
                pl.ds((j % chunks_per_row) * 16, 16)] = _threefry_core(iv)
            return iv + jnp.uint32(16)

        return lax.fori_loop(0, _SC_GRP // 16, chunk_body, iv, unroll=8)

    def dst(g):
        return bits_hbm.at[pl.ds(w_row + g * rows_per_grp, rows_per_grp)]

    def outer_body(go, iv):
        for b in range(2):
            g = go * 2 + b

            @pl.when(go > 0)
            def _():
                pltpu.make_async_copy(bufs[b], dst(g), sems[b]).wait()

            iv = fill(bufs[b], iv)
            pltpu.make_async_copy(bufs[b], dst(g), sems[b]).start()
        return iv

    lax.fori_loop(0, groups // 2, outer_body, iv0)
    for b in range(2):
        pltpu.make_async_copy(bufs[b], dst(0), sems[b]).wait()


def _sc_bits(n_tok, hidden):
    mesh = plsc.VectorSubcoreMesh(core_axis_name="c", subcore_axis_name="s")
    run = pl.kernel(
        functools.partial(_sc_bits_kernel, n_tok=n_tok, hidden=hidden),
        out_type=jax.ShapeDtypeStruct((S_SC * n_tok, hidden), jnp.uint32),
        mesh=mesh,
        scratch_types=[
            pltpu.VMEM((_SC_GRP // hidden, hidden), jnp.uint32),
            pltpu.VMEM((_SC_GRP // hidden, hidden), jnp.uint32),
            pltpu.SemaphoreType.DMA,
            pltpu.SemaphoreType.DMA,
        ],
    )
    return run()


def _tc_a_kernel(x_ref, win_ref, wout_ref, out_ref, invp_ref, psum_ref,
                 cnt_ref, *, n_tok, hidden):
    t = pl.program_id(0)
    tile_b = x_ref.shape[0]

    z = jax.lax.dot_general(
        x_ref[:], win_ref[:],
        dimension_numbers=(((1,), (1,)), ((), ())),
        preferred_element_type=jnp.float32,
    )
    zc = jnp.clip(z, -10.0, 10.0)
    p = jax.nn.sigmoid(ALPHA_C * (zc - BETA_C))
    psum_ref[:] = jnp.sum(p, axis=1, keepdims=True)
    invp_ref[:] = 1.0 / (p + 1e-20)
    cnt_ref[:] = jnp.zeros((tile_b, hidden), jnp.float32)

    b_iota = jax.lax.broadcasted_iota(jnp.uint32, (tile_b, hidden), 0)
    h_iota = jax.lax.broadcasted_iota(jnp.uint32, (tile_b, hidden), 1)
    base42 = (t.astype(jnp.uint32) * jnp.uint32(tile_b) + b_iota) \
        * jnp.uint32(hidden) + h_iota + jnp.uint32(_K1)

    def s_body(s, _):
        x1 = base42 + s.astype(jnp.uint32) * jnp.uint32(n_tok * hidden)
        v = _rank_from_bits(_threefry_core(x1), invp_ref[:])
        m = jnp.max(v, axis=1, keepdims=True)
        cnt_ref[:] += (v == m).astype(jnp.float32)
        return 0

    jax.lax.fori_loop(S_SC, S_SAMPLES, s_body, 0, unroll=False)

    acc = jnp.dot(cnt_ref[:], wout_ref[:], preferred_element_type=jnp.float32)
    out_ref[:] = acc * (psum_ref[:] * (1.0 / S_SAMPLES))


def _tc_b_kernel(bits_ref, invp_ref, psum_ref, outa_ref, wout_ref, out_ref,
                 cnt_ref):
    s = pl.program_id(1)
    tile_b = bits_ref.shape[0]
    hidden = bits_ref.shape[1]

    @pl.when(s == 0)
    def _():
        cnt_ref[:] = jnp.zeros((tile_b, hidden), jnp.float32)

    v = _rank_from_bits(bits_ref[:], invp_ref[:])
    m = jnp.max(v, axis=1, keepdims=True)
    cnt_ref[:] += (v == m).astype(jnp.float32)

    @pl.when(s == S_SC - 1)
    def _():
        acc = jnp.dot(cnt_ref[:], wout_ref[:],
                      preferred_element_type=jnp.float32)
        out_ref[:] = outa_ref[:] + acc * (psum_ref[:] * (1.0 / S_SAMPLES))


def kernel(x, in_weight, out_weight):
    n_tok, in_dim = x.shape
    hidden, out_dim = out_weight.shape
    tile_b = min(TILE_B, n_tok)
    n_tiles = n_tok // tile_b

    bits_sc = _sc_bits(n_tok, hidden)

    out_a, invp, psum = pl.pallas_call(
        functools.partial(_tc_a_kernel, n_tok=n_tok, hidden=hidden),
        grid=(n_tiles,),
        in_specs=[
            pl.BlockSpec((tile_b, in_dim), lambda t: (t, 0)),
            pl.BlockSpec((hidden, in_dim), lambda t: (0, 0)),
            pl.BlockSpec((hidden, out_dim), lambda t: (0, 0)),
        ],
        out_specs=[
            pl.BlockSpec((tile_b, out_dim), lambda t: (t, 0)),
            pl.BlockSpec((tile_b, hidden), lambda t: (t, 0)),
            pl.BlockSpec((tile_b, 1), lambda t: (t, 0)),
        ],
        out_shape=[
            jax.ShapeDtypeStruct((n_tok, out_dim), jnp.float32),
            jax.ShapeDtypeStruct((n_tok, hidden), jnp.float32),
            jax.ShapeDtypeStruct((n_tok, 1), jnp.float32),
        ],
        scratch_shapes=[
            pltpu.VMEM((tile_b, hidden), jnp.float32),
        ],
    )(x, in_weight, out_weight)

    blocks_per_s = n_tok // tile_b
    out = pl.pallas_call(
        _tc_b_kernel,
        grid=(n_tiles, S_SC),
        in_specs=[
            pl.BlockSpec((tile_b, hidden),
                         lambda t, s: (s * blocks_per_s + t, 0)),
            pl.BlockSpec((tile_b, hidden), lambda t, s: (t, 0)),
            pl.BlockSpec((tile_b, 1), lambda t, s: (t, 0)),
            pl.BlockSpec((tile_b, out_dim), lambda t, s: (t, 0)),
            pl.BlockSpec((hidden, out_dim), lambda t, s: (0, 0)),
        ],
        out_specs=pl.BlockSpec((tile_b, out_dim), lambda t, s: (t, 0)),
        out_shape=jax.ShapeDtypeStruct((n_tok, out_dim), jnp.float32),
        scratch_shapes=[
            pltpu.VMEM((tile_b, hidden), jnp.float32),
        ],
    )(bits_sc, invp, psum, out_a, out_weight)
    return out


# speedup vs baseline: 1.0114x; 1.0114x over previous
import functools

import jax
import jax.numpy as jnp
from jax import lax
from jax.experimental import pallas as pl
from jax.experimental.pallas import tpu as pltpu
from jax.experimental.pallas import tpu_sc as plsc

ALPHA_C = 5.0
BETA_C = 0.5
S_SAMPLES = 128
S_SC = 40
TILE_B = 256
_SC_WORKERS = 32
_SC_GRP = 16384

_K0 = 0
_K1 = 42
_K2 = 0x1BD11BDA ^ _K0 ^ _K1


def _rotl(v, r):
    return (v << jnp.uint32(r)) | (v >> jnp.uint32(32 - r))


def _rounds(x0, x1, rots):
    for r in rots:
        x0 = x0 + x1
        x1 = _rotl(x1, r) ^ x0
    return x0, x1


def _threefry_core(x1):
    r0 = (13, 15, 26, 6)
    r1 = (17, 29, 16, 24)
    x0 = x1
    x1 = _rotl(x1, 13) ^ x0
    x0, x1 = _rounds(x0, x1, r0[1:])
    x0 = x0 + jnp.uint32(_K1)
    x1 = x1 + jnp.uint32((_K2 + 1) & 0xFFFFFFFF)
    x0, x1 = _rounds(x0, x1, r1)
    x0 = x0 + jnp.uint32(_K2)
    x1 = x1 + jnp.uint32((_K0 + 2) & 0xFFFFFFFF)
    x0, x1 = _rounds(x0, x1, r0)
    x0 = x0 + jnp.uint32(_K0)
    x1 = x1 + jnp.uint32((_K1 + 3) & 0xFFFFFFFF)
    x0, x1 = _rounds(x0, x1, r1)
    x0 = x0 + jnp.uint32(_K1)
    x1 = x1 + jnp.uint32((_K2 + 4) & 0xFFFFFFFF)
    x0, x1 = _rounds(x0, x1, r0)
    x0 = x0 + jnp.uint32(_K2)
    x1 = x1 + jnp.uint32((_K0 + 5) & 0xFFFFFFFF)
    return x0 ^ x1


def _rank_from_bits(bits, invp):
    fb = (bits >> jnp.uint32(9)) | jnp.uint32(0x3F800000)
    u = jax.lax.bitcast_convert_type(fb, jnp.float32) - 1.0
    return jnp.log(u) * invp


def _sc_bits_kernel(bits_hbm, buf0, buf1, sem0, sem1, *, n_tok, hidden):
    wid = lax.axis_index("s") * 2 + lax.axis_index("c")
    rows_per_w = (S_SC * n_tok) // _SC_WORKERS
    rows_per_grp = _SC_GRP // hidden
    groups = rows_per_w // rows_per_grp
    chunks_per_row = hidden // 16
    w_row = wid * rows_per_w
    lane = lax.iota(jnp.int32, 16).astype(jnp.uint32)
    iv0 = (w_row * hidden).astype(jnp.uint32) + lane + jnp.uint32(_K1)
    bufs = (buf0, buf1)
    sems = (sem0, sem1)

    def fill(buf, iv):
        def chunk_body(j, iv):
            buf[j // chunks_per_row,
                pl.ds((j % chunks_per_row) * 16, 16)] = _threefry_core(iv)
            return iv + jnp.uint32(16)

        return lax.fori_loop(0, _SC_GRP // 16, chunk_body, iv, unroll=8)

    def dst(g):
        return bits_hbm.at[pl.ds(w_row + g * rows_per_grp, rows_per_grp)]

    def outer_body(go, iv):
        for b in range(2):
            g = go * 2 + b

            @pl.when(go > 0)
            def _():
                pltpu.make_async_copy(bufs[b], dst(g), sems[b]).wait()

            iv = fill(bufs[b], iv)
            pltpu.make_async_copy(bufs[b], dst(g), sems[b]).start()
        return iv

    lax.fori_loop(0, groups // 2, outer_body, iv0)
    for b in range(2):
        pltpu.make_async_copy(bufs[b], dst(0), sems[b]).wait()


def _sc_bits(n_tok, hidden):
    mesh = plsc.VectorSubcoreMesh(core_axis_name="c", subcore_axis_name="s")
    run = pl.kernel(
        functools.partial(_sc_bits_kernel, n_tok=n_tok, hidden=hidden),
        out_type=jax.ShapeDtypeStruct((S_SC * n_tok, hidden), jnp.uint32),
        mesh=mesh,
        scratch_types=[
            pltpu.VMEM((_SC_GRP // hidden, hidden), jnp.uint32),
            pltpu.VMEM((_SC_GRP // hidden, hidden), jnp.uint32),
            pltpu.SemaphoreType.DMA,
            pltpu.SemaphoreType.DMA,
        ],
    )
    return run()


def _tc_a_kernel(x_ref, win_ref, wout_ref, out_ref, invp_ref, psum_ref,
                 cnt_ref, *, n_tok, hidden):
    t = pl.program_id(0)
    tile_b = x_ref.shape[0]

    z = jax.lax.dot_general(
        x_ref[:], win_ref[:],
        dimension_numbers=(((1,), (1,)), ((), ())),
        preferred_element_type=jnp.float32,
    )
    zc = jnp.clip(z, -10.0, 10.0)
    p = jax.nn.sigmoid(ALPHA_C * (zc - BETA_C))
    psum_ref[:] = jnp.sum(p, axis=1, keepdims=True)
    invp_ref[:] = 1.0 / (p + 1e-20)
    cnt_ref[:] = jnp.zeros((tile_b, hidden), jnp.float32)

    b_iota = jax.lax.broadcasted_iota(jnp.uint32, (tile_b, hidden), 0)
    h_iota = jax.lax.broadcasted_iota(jnp.uint32, (tile_b, hidden), 1)
    base42 = (t.astype(jnp.uint32) * jnp.uint32(tile_b) + b_iota) \
        * jnp.uint32(hidden) + h_iota + jnp.uint32(_K1)

    def s_body(s, _):
        x1 = base42 + s.astype(jnp.uint32) * jnp.uint32(n_tok * hidden)
        v = _rank_from_bits(_threefry_core(x1), invp_ref[:])
        m = jnp.max(v, axis=1, keepdims=True)
        cnt_ref[:] += (v == m).astype(jnp.float32)
        return 0

    jax.lax.fori_loop(S_SC, S_SAMPLES, s_body, 0, unroll=False)

    acc = jnp.dot(cnt_ref[:], wout_ref[:], preferred_element_type=jnp.float32)
    out_ref[:] = acc * (psum_ref[:] * (1.0 / S_SAMPLES))


def _tc_b_kernel(bits_ref, invp_ref, psum_ref, outa_ref, wout_ref, out_ref,
                 cnt_ref):
    s = pl.program_id(1)
    tile_b = bits_ref.shape[0]
    hidden = bits_ref.shape[1]

    @pl.when(s == 0)
    def _():
        cnt_ref[:] = jnp.zeros((tile_b, hidden), jnp.float32)

    v = _rank_from_bits(bits_ref[:], invp_ref[:])
    m = jnp.max(v, axis=1, keepdims=True)
    cnt_ref[:] += (v == m).astype(jnp.float32)

    @pl.when(s == S_SC - 1)
    def _():
        acc = jnp.dot(cnt_ref[:], wout_ref[:],
                      preferred_element_type=jnp.float32)
        out_ref[:] = outa_ref[:] + acc * (psum_ref[:] * (1.0 / S_SAMPLES))


def kernel(x, in_weight, out_weight):
    n_tok, in_dim = x.shape
    hidden, out_dim = out_weight.shape
    tile_b = min(TILE_B, n_tok)
    n_tiles = n_tok // tile_b

    bits_sc = _sc_bits(n_tok, hidden)

    out_a, invp, psum = pl.pallas_call(
        functools.partial(_tc_a_kernel, n_tok=n_tok, hidden=hidden),
        grid=(n_tiles,),
        in_specs=[
            pl.BlockSpec((tile_b, in_dim), lambda t: (t, 0)),
            pl.BlockSpec((hidden, in_dim), lambda t: (0, 0)),
            pl.BlockSpec((hidden, out_dim), lambda t: (0, 0)),
        ],
        out_specs=[
            pl.BlockSpec((tile_b, out_dim), lambda t: (t, 0)),
            pl.BlockSpec((tile_b, hidden), lambda t: (t, 0)),
            pl.BlockSpec((tile_b, 1), lambda t: (t, 0)),
        ],
        out_shape=[
            jax.ShapeDtypeStruct((n_tok, out_dim), jnp.float32),
            jax.ShapeDtypeStruct((n_tok, hidden), jnp.float32),
            jax.ShapeDtypeStruct((n_tok, 1), jnp.float32),
        ],
        scratch_shapes=[
            pltpu.VMEM((tile_b, hidden), jnp.float32),
        ],
    )(x, in_weight, out_weight)

    blocks_per_s = n_tok // tile_b
    out = pl.pallas_call(
        _tc_b_kernel,
        grid=(n_tiles, S_SC),
        in_specs=[
            pl.BlockSpec((tile_b, hidden),
                         lambda t, s: (s * blocks_per_s + t, 0)),
            pl.BlockSpec((tile_b, hidden), lambda t, s: (t, 0)),
            pl.BlockSpec((tile_b, 1), lambda t, s: (t, 0)),
            pl.BlockSpec((tile_b, out_dim), lambda t, s: (t, 0)),
            pl.BlockSpec((hidden, out_dim), lambda t, s: (0, 0)),
        ],
        out_specs=pl.BlockSpec((tile_b, out_dim), lambda t, s: (t, 0)),
        out_shape=jax.ShapeDtypeStruct((n_tok, out_dim), jnp.float32),
        scratch_shapes=[
            pltpu.VMEM((tile_b, hidden), jnp.float32),
        ],
    )(bits_sc, invp, psum, out_a, out_weight)
    return out
